# Initial kernel scaffold; baseline (speedup 1.0000x reference)
import jax
import jax.numpy as jnp
from jax.experimental import pallas as pl


def kernel(input_var, encoder_outputs, emb, W_ih, W_hh, b_ih, b_hh, W_c, W_out):
    raise NotImplementedError("write your pallas kernel here")


# speedup vs baseline: 6.4666x; 6.4666x over previous
import functools

import jax
import jax.numpy as jnp
from jax import lax
from jax.experimental import pallas as pl
from jax.experimental.pallas import tpu as pltpu
from jax.experimental.pallas import tpu_sc as plsc

B = 16
K = 8
V = 4096
H = 1024
S = 256
T = 8
BK = B * K

_NEG_INF = float("-inf")
_BIG = 2**30

_SC_NC = 2
_SC_WORKERS = 16


def _decode_body(emb_ref, encT_ref, wih_ref, whh_ref, bih_ref,
                 bhh_ref, wc_ref, wout_ref, outs_ref, preds_ref, idx_ref,
                 so_buf, dma_sem):
    f32 = jnp.float32
    bf = jnp.bfloat16

    row = lax.broadcasted_iota(jnp.int32, (BK, 1), 0)
    bcol = row % B
    g = (8 * (row % B) + row // B) % B
    lane_v = lax.broadcasted_iota(jnp.int32, (BK, V), 1)
    att_mask = (lane_v // S) == g
    code16 = lax.broadcasted_iota(jnp.int32, (B, V), 1)

    bih = bih_ref[...]
    bhh = bhh_ref[...]

    def step(t, carry):
        ids, h, score = carry
        oh = (ids == lane_v).astype(bf)
        x_bf = jnp.dot(oh, emb_ref[...], preferred_element_type=f32).astype(bf)

        gi = jnp.dot(x_bf, wih_ref[...], preferred_element_type=f32) + bih
        gh = jnp.dot(h.astype(bf), whh_ref[...], preferred_element_type=f32) + bhh
        r = jax.nn.sigmoid(gi[:, :H] + gh[:, :H])
        z = jax.nn.sigmoid(gi[:, H:2 * H] + gh[:, H:2 * H])
        n = jnp.tanh(gi[:, 2 * H:] + r * gh[:, 2 * H:])
        h_new = (1.0 - z) * n + z * h

        hnb = h_new.astype(bf)
        att = jnp.dot(hnb, encT_ref[...], preferred_element_type=f32) / 32.0
        att = jnp.where(att_mask, att, _NEG_INF)
        am = jnp.max(att, axis=1, keepdims=True)
        ex = jnp.exp(att - am)
        al = ex / jnp.sum(ex, axis=1, keepdims=True)
        ctx = lax.dot_general(al.astype(bf), encT_ref[...],
                              (((1,), (1,)), ((), ())),
                              preferred_element_type=f32)

        cc = jnp.concatenate([hnb, ctx.astype(bf)], axis=1)
        comb = jnp.tanh(jnp.dot(cc, wc_ref[...], preferred_element_type=f32))
        logits = jnp.dot(comb.astype(bf), wout_ref[...],
                         preferred_element_type=f32)
        lm = jnp.max(logits, axis=1, keepdims=True)
        lse = jnp.log(jnp.sum(jnp.exp(logits - lm), axis=1, keepdims=True))
        so = logits - lm - lse
        so_buf[...] = so
        cp = pltpu.make_async_copy(so_buf, outs_ref.at[t], dma_sem)
        cp.start()
        cp.wait()

        seq = score + so
        sl = [seq[j * B:(j + 1) * B, :] for j in range(K)]
        scs, fis = [], []
        for _ in range(K):
            gmax = sl[0]
            for j in range(1, K):
                gmax = jnp.maximum(gmax, sl[j])
            m = jnp.max(gmax, axis=1, keepdims=True)
            fi = None
            for j in range(K):
                cj = jnp.where(sl[j] == m, code16 + j * V, _BIG)
                mn = jnp.min(cj, axis=1, keepdims=True)
                fi = mn if fi is None else jnp.minimum(fi, mn)
            scs.append(m)
            fis.append(fi)
            for j in range(K):
                sl[j] = jnp.where(code16 + j * V == fi, _NEG_INF, sl[j])

        score_new = jnp.concatenate(scs, axis=0)
        fi_col = jnp.concatenate(fis, axis=0)
        ids_new = fi_col % V
        preds_ref[t] = (fi_col // V) * B + bcol

        hs = []
        for nblk in range(K):
            jp_col = fis[nblk] // V
            acc = h_new[0:B, :]
            for jp in range(1, K):
                acc = jnp.where(jp_col == jp, h_new[jp * B:(jp + 1) * B, :],
                                acc)
            hs.append(acc)
        return ids_new, jnp.concatenate(hs, axis=0), score_new

    ids0 = jnp.zeros((BK, 1), jnp.int32)
    h0 = jnp.zeros((BK, H), f32)
    score0 = jnp.where(row < B, 0.0, _NEG_INF).astype(f32)
    _, _, score = lax.fori_loop(0, T, step, (ids0, h0, score0))

    best = score[0:B]
    bestj = jnp.zeros((B, 1), jnp.int32)
    for j in range(1, K):
        v = score[j * B:(j + 1) * B]
        upd = v > best
        bestj = jnp.where(upd, j, bestj)
        best = jnp.where(upd, v, best)
    iota_b = lax.broadcasted_iota(jnp.int32, (B, 1), 0)
    lane128_b = lax.broadcasted_iota(jnp.int32, (B, BK), 1)
    cur = bestj * B + iota_b
    for t in range(T - 1, -1, -1):
        idx_ref[t * B:(t + 1) * B] = cur + t * BK
        ohc = (cur == lane128_b).astype(bf)
        pf = preds_ref[t].astype(bf)
        cur = jnp.dot(ohc, pf, preferred_element_type=jnp.float32
                      ).astype(jnp.int32)


def _sc_gather_body(outs_hbm, idx_hbm, out_hbm, idx_v, rows_v, sem):
    wid = lax.axis_index("s") * _SC_NC + lax.axis_index("c")

    @pl.when(wid < _SC_WORKERS)
    def _():
        base = pl.multiple_of(wid * K, 8)
        pltpu.sync_copy(idx_hbm.at[pl.ds(base, K)], idx_v)
        pltpu.async_copy(outs_hbm.at[idx_v], rows_v, sem).wait()
        pltpu.sync_copy(rows_v, out_hbm.at[pl.ds(base, K)])


@functools.lru_cache(maxsize=1)
def _sc_gather():
    return pl.kernel(
        _sc_gather_body,
        out_type=jax.ShapeDtypeStruct((BK, V), jnp.float32),
        mesh=plsc.VectorSubcoreMesh(core_axis_name="c", subcore_axis_name="s",
                                    num_cores=_SC_NC, num_subcores=16),
        scratch_types=[
            pltpu.VMEM((K,), jnp.int32),
            pltpu.VMEM((K, V), jnp.float32),
            pltpu.SemaphoreType.DMA,
        ],
    )


def _decode(enc_all, emb, W_ih, W_hh, b_ih, b_hh, W_c, W_out):
    bf = jnp.bfloat16
    outs, _, idx = pl.pallas_call(
        _decode_body,
        out_shape=[
            jax.ShapeDtypeStruct((T, BK, V), jnp.float32),
            jax.ShapeDtypeStruct((T, BK, 1), jnp.int32),
            jax.ShapeDtypeStruct((BK, 1), jnp.int32),
        ],
        out_specs=[
            pl.BlockSpec(memory_space=pl.ANY),
            pl.BlockSpec(memory_space=pltpu.VMEM),
            pl.BlockSpec(memory_space=pltpu.VMEM),
        ],
        scratch_shapes=[
            pltpu.VMEM((BK, V), jnp.float32),
            pltpu.SemaphoreType.DMA,
        ],
        compiler_params=pltpu.CompilerParams(
            vmem_limit_bytes=62 * 1024 * 1024),
    )(
        emb.astype(bf),
        enc_all.T.astype(bf),
        W_ih.T.astype(bf),
        W_hh.T.astype(bf),
        b_ih.reshape(1, 3 * H),
        b_hh.reshape(1, 3 * H),
        W_c.T.astype(bf),
        W_out.T.astype(bf),
    )
    return outs, idx


def kernel(input_var, encoder_outputs, emb, W_ih, W_hh, b_ih, b_hh, W_c,
           W_out):
    del input_var
    enc_all = encoder_outputs.reshape(B * S, H)
    outs, idx = _decode(enc_all, emb, W_ih, W_hh, b_ih, b_hh, W_c, W_out)
    final = _sc_gather()(outs.reshape(T * BK, V), idx.reshape(BK))
    return final.reshape(T, B, V)
